# Initial kernel scaffold; baseline (speedup 1.0000x reference)
import jax
import jax.numpy as jnp
from jax.experimental import pallas as pl


def kernel(input, adj, W, b, gamma, beta):
    raise NotImplementedError("write your pallas kernel here")


# speedup vs baseline: 1.0451x; 1.0451x over previous
import jax
import jax.numpy as jnp
from jax.experimental import pallas as pl
from jax.experimental.pallas import tpu as pltpu

N = 10000
D_IN = 128
D_OUT = 128
BM = 400


def _gcn_body(adj_ref, x_ref, w_ref, g_ref, bta_ref, out_ref, sup_ref):
    i = pl.program_id(0)

    @pl.when(i == 0)
    def _():
        sup_ref[...] = jnp.dot(
            x_ref[...], w_ref[...], preferred_element_type=jnp.float32
        )

    out_ref[pl.ds(i * BM, BM), :] = jnp.dot(
        adj_ref[...], sup_ref[...], preferred_element_type=jnp.float32
    )

    @pl.when(i == pl.num_programs(0) - 1)
    def _():
        y = out_ref[...]
        mean = jnp.mean(y, axis=0, keepdims=True)
        yc = y - mean
        var = jnp.mean(yc * yc, axis=0, keepdims=True)
        z = yc * jax.lax.rsqrt(var + 1e-5) * g_ref[...] + bta_ref[...]
        out_ref[...] = jnp.where(z >= 0, z, 0.01 * z)


def kernel(input, adj, W, b, gamma, beta):
    del b
    g2 = gamma.reshape(1, D_OUT)
    bt2 = beta.reshape(1, D_OUT)
    grid = (N // BM,)
    return pl.pallas_call(
        _gcn_body,
        grid=grid,
        in_specs=[
            pl.BlockSpec((BM, N), lambda i: (i, 0)),
            pl.BlockSpec((N, D_IN), lambda i: (0, 0)),
            pl.BlockSpec((D_IN, D_OUT), lambda i: (0, 0)),
            pl.BlockSpec((1, D_OUT), lambda i: (0, 0)),
            pl.BlockSpec((1, D_OUT), lambda i: (0, 0)),
        ],
        out_specs=pl.BlockSpec((N, D_OUT), lambda i: (0, 0)),
        out_shape=jax.ShapeDtypeStruct((N, D_OUT), jnp.float32),
        scratch_shapes=[pltpu.VMEM((N, D_IN), jnp.float32)],
    )(adj, input, W, g2, bt2)


# speedup vs baseline: 1.0582x; 1.0126x over previous
import jax
import jax.numpy as jnp
from jax.experimental import pallas as pl
from jax.experimental.pallas import tpu as pltpu

N = 10000
D_IN = 128
D_OUT = 128
BM = 200


def _gcn_body(adj_ref, x_ref, w_ref, g_ref, bta_ref, out_ref, sup_ref):
    i = pl.program_id(0)

    @pl.when(i == 0)
    def _():
        sup_ref[...] = jnp.dot(
            x_ref[...], w_ref[...], preferred_element_type=jnp.float32
        )

    out_ref[pl.ds(i * BM, BM), :] = jnp.dot(
        adj_ref[...], sup_ref[...], preferred_element_type=jnp.float32
    )

    @pl.when(i == pl.num_programs(0) - 1)
    def _():
        y = out_ref[...]
        mean = jnp.mean(y, axis=0, keepdims=True)
        yc = y - mean
        var = jnp.mean(yc * yc, axis=0, keepdims=True)
        z = yc * jax.lax.rsqrt(var + 1e-5) * g_ref[...] + bta_ref[...]
        out_ref[...] = jnp.where(z >= 0, z, 0.01 * z)


def kernel(input, adj, W, b, gamma, beta):
    del b
    g2 = gamma.reshape(1, D_OUT)
    bt2 = beta.reshape(1, D_OUT)
    grid = (N // BM,)
    return pl.pallas_call(
        _gcn_body,
        grid=grid,
        in_specs=[
            pl.BlockSpec((BM, N), lambda i: (i, 0)),
            pl.BlockSpec((N, D_IN), lambda i: (0, 0)),
            pl.BlockSpec((D_IN, D_OUT), lambda i: (0, 0)),
            pl.BlockSpec((1, D_OUT), lambda i: (0, 0)),
            pl.BlockSpec((1, D_OUT), lambda i: (0, 0)),
        ],
        out_specs=pl.BlockSpec((N, D_OUT), lambda i: (0, 0)),
        out_shape=jax.ShapeDtypeStruct((N, D_OUT), jnp.float32),
        scratch_shapes=[pltpu.VMEM((N, D_IN), jnp.float32)],
    )(adj, input, W, g2, bt2)
